# Initial kernel scaffold; baseline (speedup 1.0000x reference)
import jax
import jax.numpy as jnp
from jax.experimental import pallas as pl


def kernel(x, m28, m14, m7, sel28, sel14, up7, up14, avg7, bc7, w00, w01, w02, w03, w04, w05, w06, w07, w08, w09, w10, w11, w12, w13, w14, w15, w16, w17, w18, w19, w20, w21, w22, w23, w24, w25, w26, w27, w28, w29, w30, w31):
    raise NotImplementedError("write your pallas kernel here")


# speedup vs baseline: 1.4983x; 1.4983x over previous
import numpy as np

import jax
import jax.numpy as jnp
from jax.experimental import pallas as pl
from jax.experimental.pallas import tpu as pltpu

_HALO = 64

_T28 = (29, 32, 28, 28)
_T14 = (15, 16, 14, 14)
_T7 = (8, 8, 7, 7)


def _flat(t):
    return t[0] * t[1]


def _gap_mask(t, B):
    th, tw, h, w = t
    m = np.zeros((th, tw), np.float32)
    m[1:1 + h, 0:w] = 1.0
    return np.tile(m.reshape(1, -1), (1, B))


def _pool_sel(t_in, t_out):
    ti_h, ti_w, h, w = t_in
    to_h, to_w, h2, w2 = t_out
    S = np.zeros((ti_h * ti_w, to_h * to_w), np.float32)
    for y2 in range(h2):
        for x2 in range(w2):
            q = (1 + y2) * to_w + x2
            p = (1 + 2 * y2) * ti_w + 2 * x2
            S[p, q] = 1.0
    return S


def _upsample(t_in, t_out):
    ti_h, ti_w, h, w = t_in
    to_h, to_w, h2, w2 = t_out
    U = np.zeros((ti_h * ti_w, to_h * to_w), np.float32)
    for y2 in range(h2):
        for x2 in range(w2):
            q = (1 + y2) * to_w + x2
            p = (1 + y2 // 2) * ti_w + x2 // 2
            U[p, q] = 1.0
    return U


def _avg_mat(B):
    f7 = _flat(_T7)
    g = _gap_mask(_T7, 1).reshape(-1)
    A = np.zeros((B * f7, B), np.float32)
    for b in range(B):
        A[b * f7:(b + 1) * f7, b] = g / 49.0
    return A


def _bc_mat(B):
    f7 = _flat(_T7)
    g = _gap_mask(_T7, 1).reshape(-1)
    M = np.zeros((B, B * f7), np.float32)
    for b in range(B):
        M[b, b * f7:(b + 1) * f7] = g
    return M


def _ae_kernel(
    x_ref,
    g28_ref, g14_ref, g7_ref,
    s28_ref, s14_ref, u7_ref, u14_ref, avg_ref, bc_ref,
    we1_ref, be1_ref, we2_ref, be2_ref, we3_ref, be3_ref,
    we4_ref, be4_ref, we5_ref, be5_ref, we6_ref, be6_ref,
    wl1_ref, bl1_ref, wl2_ref, bl2_ref,
    wd1_ref, bd1_ref, wd2_ref, bd2_ref,
    wc0_ref, bc0_ref, wc1_ref, bc1_ref, wc2_ref, bc2_ref,
    wc3_ref, bc3_ref, wc4_ref, bc4_ref, wc5_ref, bc5_ref,
    enc_ref, loss_ref,
    bufa, bufb,
):
    H = _HALO
    B = enc_ref.shape[1]
    F28, F14, F7 = _flat(_T28), _flat(_T14), _flat(_T7)
    N28, N14, N7 = B * F28, B * F14, B * F7
    W28, W14, W7 = _T28[1], _T14[1], _T7[1]

    bufa[...] = jnp.zeros_like(bufa)
    bufb[...] = jnp.zeros_like(bufb)

    xv = x_ref[...]
    bufa[0:1, H:H + N28] = xv

    def conv3x3(src, dst, w_ref, b_ref, cin, cout, Wp, N, act, gmask_ref):
        acc = jnp.zeros((cout, N), jnp.float32)
        for ky in range(3):
            for kx in range(3):
                d = (ky - 1) * Wp + (kx - 1)
                win = src[0:cin, H + d:H + d + N]
                wk = w_ref[ky * 3 + kx]
                if cin <= 2:
                    for ci in range(cin):
                        acc = acc + wk[:, ci:ci + 1] * win[ci:ci + 1, :]
                else:
                    acc = acc + jnp.dot(wk, win,
                                        preferred_element_type=jnp.float32)
        acc = acc + b_ref[...]
        if act == "relu":
            acc = jnp.maximum(acc, 0.0)
        elif act == "tanh":
            acc = jnp.tanh(acc)
        if gmask_ref is not None:
            acc = acc * gmask_ref[...]
        dst[0:cout, H:H + N] = acc
        dst[0:cout, H + N:H + N + H] = jnp.zeros((cout, H), jnp.float32)

    def maxpool2x2(src, dst, s_ref, c, Wp, Fin, Fout, Nin):
        m = src[0:c, H:H + Nin]
        for d in (1, Wp, Wp + 1):
            m = jnp.maximum(m, src[0:c, H + d:H + d + Nin])
        for b in range(B):
            blk = jnp.dot(m[:, b * Fin:(b + 1) * Fin], s_ref[...],
                          preferred_element_type=jnp.float32)
            dst[0:c, H + b * Fout:H + (b + 1) * Fout] = blk
        dst[0:c, H + B * Fout:H + B * Fout + H] = jnp.zeros((c, H), jnp.float32)

    def up2x_relu(src, dst, u_ref, c, Fin, Fout):
        for b in range(B):
            blk = jnp.dot(src[0:c, H + b * Fin:H + (b + 1) * Fin], u_ref[...],
                          preferred_element_type=jnp.float32)
            dst[0:c, H + b * Fout:H + (b + 1) * Fout] = jnp.maximum(blk, 0.0)
        dst[0:c, H + B * Fout:H + B * Fout + H] = jnp.zeros((c, H), jnp.float32)

    conv3x3(bufa, bufb, we1_ref, be1_ref, 1, 2, W28, N28, None, g28_ref)
    conv3x3(bufb, bufa, we2_ref, be2_ref, 2, 4, W28, N28, "relu", None)
    maxpool2x2(bufa, bufb, s28_ref, 4, W28, F28, F14, N28)
    conv3x3(bufb, bufa, we3_ref, be3_ref, 4, 8, W14, N14, None, g14_ref)
    conv3x3(bufa, bufb, we4_ref, be4_ref, 8, 8, W14, N14, "relu", None)
    maxpool2x2(bufb, bufa, s14_ref, 8, W14, F14, F7, N14)
    conv3x3(bufa, bufb, we5_ref, be5_ref, 8, 16, W7, N7, None, g7_ref)
    conv3x3(bufb, bufa, we6_ref, be6_ref, 16, 32, W7, N7, "relu", None)

    pooled = jnp.dot(bufa[0:32, H:H + N7], avg_ref[...],
                     preferred_element_type=jnp.float32)
    z1 = jnp.maximum(
        jnp.dot(wl1_ref[...], pooled, preferred_element_type=jnp.float32)
        + bl1_ref[...], 0.0)
    enc = (jnp.dot(wl2_ref[...], z1, preferred_element_type=jnp.float32)
           + bl2_ref[...])
    enc_ref[...] = enc

    d1 = jnp.maximum(
        jnp.dot(wd1_ref[...], enc, preferred_element_type=jnp.float32)
        + bd1_ref[...], 0.0)
    d2 = (jnp.dot(wd2_ref[...], d1, preferred_element_type=jnp.float32)
          + bd2_ref[...])
    d3 = jnp.maximum(
        jnp.dot(d2, bc_ref[...], preferred_element_type=jnp.float32), 0.0)
    bufb[0:32, H:H + N7] = d3
    bufb[0:32, H + N7:H + N7 + H] = jnp.zeros((32, H), jnp.float32)

    conv3x3(bufb, bufa, wc0_ref, bc0_ref, 32, 16, W7, N7, None, g7_ref)
    conv3x3(bufa, bufb, wc1_ref, bc1_ref, 16, 8, W7, N7, None, None)
    up2x_relu(bufb, bufa, u7_ref, 8, F7, F14)
    conv3x3(bufa, bufb, wc2_ref, bc2_ref, 8, 8, W14, N14, None, g14_ref)
    conv3x3(bufb, bufa, wc3_ref, bc3_ref, 8, 4, W14, N14, None, None)
    up2x_relu(bufa, bufb, u14_ref, 4, F14, F28)
    conv3x3(bufb, bufa, wc4_ref, bc4_ref, 4, 2, W28, N28, None, g28_ref)
    conv3x3(bufa, bufb, wc5_ref, bc5_ref, 2, 1, W28, N28, "tanh", g28_ref)

    decoded = bufb[0:1, H:H + N28]
    diff = decoded - xv
    loss_ref[...] = jnp.sum(diff * diff, axis=1, keepdims=True) / float(784 * B)


def _zero_map(nd):
    return lambda i: (0,) * nd


def kernel(x, m28, m14, m7, sel28, sel14, up7, up14, avg7, bc7,
           w00, w01, w02, w03, w04, w05, w06, w07, w08, w09,
           w10, w11, w12, w13, w14, w15, w16, w17, w18, w19,
           w20, w21, w22, w23, w24, w25, w26, w27, w28, w29,
           w30, w31):
    B = x.shape[0]
    F28 = _flat(_T28)
    N28 = B * F28

    xp = jnp.pad(x, ((0, 0), (1, 0), (0, 4))).reshape(1, N28)

    consts = [
        jnp.asarray(_gap_mask(_T28, B)), jnp.asarray(_gap_mask(_T14, B)),
        jnp.asarray(_gap_mask(_T7, B)),
        jnp.asarray(_pool_sel(_T28, _T14)), jnp.asarray(_pool_sel(_T14, _T7)),
        jnp.asarray(_upsample(_T7, _T14)), jnp.asarray(_upsample(_T14, _T28)),
        jnp.asarray(_avg_mat(B)), jnp.asarray(_bc_mat(B)),
    ]
    weights = [w00, w01, w02, w03, w04, w05, w06, w07, w08, w09,
               w10, w11, w12, w13, w14, w15, w16, w17, w18, w19,
               w20, w21, w22, w23, w24, w25, w26, w27, w28, w29,
               w30, w31]
    args = [xp] + consts + weights

    buf_w = 2 * _HALO + N28

    enc, loss = pl.pallas_call(
        _ae_kernel,
        grid=(1,),
        in_specs=[pl.BlockSpec(a.shape, _zero_map(a.ndim)) for a in args],
        out_specs=(pl.BlockSpec((16, B), lambda i: (0, 0)),
                   pl.BlockSpec((1, 1), lambda i: (0, 0))),
        out_shape=(jax.ShapeDtypeStruct((16, B), jnp.float32),
                   jax.ShapeDtypeStruct((1, 1), jnp.float32)),
        scratch_shapes=[
            pltpu.VMEM((32, buf_w), jnp.float32),
            pltpu.VMEM((32, buf_w), jnp.float32),
        ],
        compiler_params=pltpu.CompilerParams(
            dimension_semantics=("arbitrary",),
            vmem_limit_bytes=32 * 1024 * 1024,
        ),
        cost_estimate=pl.CostEstimate(flops=16_000_000, transcendentals=25_000,
                                      bytes_accessed=3_000_000),
    )(*args)
    return enc.T, loss[0, 0]


# speedup vs baseline: 3.0514x; 2.0365x over previous
import numpy as np

import jax
import jax.numpy as jnp
from jax.experimental import pallas as pl
from jax.experimental.pallas import tpu as pltpu

_HALO = 64

_T28 = (29, 32, 28, 28)
_T14 = (15, 16, 14, 14)
_T7 = (8, 8, 7, 7)

_CONVS = [(1, 2), (2, 4), (4, 8), (8, 8), (8, 16), (16, 32),
          (32, 16), (16, 8), (8, 8), (8, 4), (4, 2), (2, 1)]


def _rpad(n):
    return (n + 7) & ~7


def _conv_bases():
    bases, r = [], 0
    for _, co in _CONVS:
        bases.append(r)
        r += 9 * _rpad(co)
    lin_bases = []
    for rows in (32, 16, 32, 32):
        lin_bases.append(r)
        r += rows
    return bases, lin_bases, r


_CBASES, _LBASES, _WROWS = _conv_bases()


def _flat(t):
    return t[0] * t[1]


def _gap_mask(t, B):
    th, tw, h, w = t
    m = np.zeros((th, tw), np.float32)
    m[1:1 + h, 0:w] = 1.0
    return np.tile(m.reshape(1, -1), (1, B))


def _pool_sel(t_in, t_out):
    ti_h, ti_w, h, w = t_in
    to_h, to_w, h2, w2 = t_out
    S = np.zeros((ti_h * ti_w, to_h * to_w), np.float32)
    for y2 in range(h2):
        for x2 in range(w2):
            S[(1 + 2 * y2) * ti_w + 2 * x2, (1 + y2) * to_w + x2] = 1.0
    return S


def _upsample(t_in, t_out):
    ti_h, ti_w, h, w = t_in
    to_h, to_w, h2, w2 = t_out
    U = np.zeros((ti_h * ti_w, to_h * to_w), np.float32)
    for y2 in range(h2):
        for x2 in range(w2):
            U[(1 + y2 // 2) * ti_w + x2 // 2, (1 + y2) * to_w + x2] = 1.0
    return U


def _avg_mat(B):
    f7 = _flat(_T7)
    g = _gap_mask(_T7, 1).reshape(-1)
    A = np.zeros((B * f7, B), np.float32)
    for b in range(B):
        A[b * f7:(b + 1) * f7, b] = g / 49.0
    return A


def _bc_mat(B):
    f7 = _flat(_T7)
    g = _gap_mask(_T7, 1).reshape(-1)
    M = np.zeros((B, B * f7), np.float32)
    for b in range(B):
        M[b, b * f7:(b + 1) * f7] = g
    return M


def _ae_kernel(
    x_ref,
    g28_ref, g14_ref, g7_ref,
    s28_ref, s14_ref, u7_ref, u14_ref, avg_ref, bc_ref,
    wp_ref, bp_ref,
    enc_ref, loss_ref,
    bufa, bufb,
):
    H = _HALO
    B = enc_ref.shape[0]
    F28, F14, F7 = _flat(_T28), _flat(_T14), _flat(_T7)
    N28, N14, N7 = B * F28, B * F14, B * F7
    W28, W14, W7 = _T28[1], _T14[1], _T7[1]

    bufa[...] = jnp.zeros_like(bufa)
    bufb[...] = jnp.zeros_like(bufb)

    xv = x_ref[...]
    bufa[0:1, H:H + N28] = xv

    def conv3x3(src, dst, idx, Wp, N, act, gmask_ref):
        cin, cout = _CONVS[idx]
        base, cp = _CBASES[idx], _rpad(_CONVS[idx][1])
        acc = jnp.zeros((cout, N), jnp.float32)
        for ky in range(3):
            for kx in range(3):
                d = (ky - 1) * Wp + (kx - 1)
                win = src[0:cin, H + d:H + d + N]
                r0 = base + (ky * 3 + kx) * cp
                wk = wp_ref[r0:r0 + cout, 0:cin]
                if cin <= 2:
                    for ci in range(cin):
                        acc = acc + wk[:, ci:ci + 1] * win[ci:ci + 1, :]
                else:
                    acc = acc + jnp.dot(wk, win,
                                        preferred_element_type=jnp.float32)
        acc = acc + bp_ref[0:cout, idx:idx + 1]
        if act == "relu":
            acc = jnp.maximum(acc, 0.0)
        elif act == "tanh":
            acc = jnp.tanh(acc)
        if gmask_ref is not None:
            acc = acc * gmask_ref[...]
        dst[0:cout, H:H + N] = acc
        dst[0:cout, H + N:H + N + H] = jnp.zeros((cout, H), jnp.float32)

    def maxpool2x2(src, dst, s_ref, c, Wp, Fin, Fout, Nin):
        m = src[0:c, H:H + Nin]
        for d in (1, Wp, Wp + 1):
            m = jnp.maximum(m, src[0:c, H + d:H + d + Nin])
        for b in range(B):
            blk = jnp.dot(m[:, b * Fin:(b + 1) * Fin], s_ref[...],
                          preferred_element_type=jnp.float32)
            dst[0:c, H + b * Fout:H + (b + 1) * Fout] = blk
        dst[0:c, H + B * Fout:H + B * Fout + H] = jnp.zeros((c, H), jnp.float32)

    def up2x_relu(src, dst, u_ref, c, Fin, Fout):
        for b in range(B):
            blk = jnp.dot(src[0:c, H + b * Fin:H + (b + 1) * Fin], u_ref[...],
                          preferred_element_type=jnp.float32)
            dst[0:c, H + b * Fout:H + (b + 1) * Fout] = jnp.maximum(blk, 0.0)
        dst[0:c, H + B * Fout:H + B * Fout + H] = jnp.zeros((c, H), jnp.float32)

    def lin_w(j, rows, cols):
        return wp_ref[_LBASES[j]:_LBASES[j] + rows, 0:cols]

    def lin_b(j, rows):
        return bp_ref[0:rows, 12 + j:13 + j]

    conv3x3(bufa, bufb, 0, W28, N28, None, g28_ref)
    conv3x3(bufb, bufa, 1, W28, N28, "relu", None)
    maxpool2x2(bufa, bufb, s28_ref, 4, W28, F28, F14, N28)
    conv3x3(bufb, bufa, 2, W14, N14, None, g14_ref)
    conv3x3(bufa, bufb, 3, W14, N14, "relu", None)
    maxpool2x2(bufb, bufa, s14_ref, 8, W14, F14, F7, N14)
    conv3x3(bufa, bufb, 4, W7, N7, None, g7_ref)
    conv3x3(bufb, bufa, 5, W7, N7, "relu", None)

    pooled = jnp.dot(bufa[0:32, H:H + N7], avg_ref[...],
                     preferred_element_type=jnp.float32)
    z1 = jnp.maximum(
        jnp.dot(lin_w(0, 32, 32), pooled, preferred_element_type=jnp.float32)
        + lin_b(0, 32), 0.0)
    enc = (jnp.dot(lin_w(1, 16, 32), z1, preferred_element_type=jnp.float32)
           + lin_b(1, 16))
    enc_ref[...] = enc.T

    d1 = jnp.maximum(
        jnp.dot(lin_w(2, 32, 16), enc, preferred_element_type=jnp.float32)
        + lin_b(2, 32), 0.0)
    d2 = (jnp.dot(lin_w(3, 32, 32), d1, preferred_element_type=jnp.float32)
          + lin_b(3, 32))
    d3 = jnp.maximum(
        jnp.dot(d2, bc_ref[...], preferred_element_type=jnp.float32), 0.0)
    bufb[0:32, H:H + N7] = d3
    bufb[0:32, H + N7:H + N7 + H] = jnp.zeros((32, H), jnp.float32)

    conv3x3(bufb, bufa, 6, W7, N7, None, g7_ref)
    conv3x3(bufa, bufb, 7, W7, N7, None, None)
    up2x_relu(bufb, bufa, u7_ref, 8, F7, F14)
    conv3x3(bufa, bufb, 8, W14, N14, None, g14_ref)
    conv3x3(bufb, bufa, 9, W14, N14, None, None)
    up2x_relu(bufa, bufb, u14_ref, 4, F14, F28)
    conv3x3(bufb, bufa, 10, W28, N28, None, g28_ref)
    conv3x3(bufa, bufb, 11, W28, N28, "tanh", g28_ref)

    decoded = bufb[0:1, H:H + N28]
    diff = decoded - xv
    loss_ref[...] = jnp.sum(diff * diff, axis=1, keepdims=True) / float(784 * B)


def _zero_map(nd):
    return lambda i: (0,) * nd


def kernel(x, m28, m14, m7, sel28, sel14, up7, up14, avg7, bc7,
           w00, w01, w02, w03, w04, w05, w06, w07, w08, w09,
           w10, w11, w12, w13, w14, w15, w16, w17, w18, w19,
           w20, w21, w22, w23, w24, w25, w26, w27, w28, w29,
           w30, w31):
    B = x.shape[0]
    N28 = B * _flat(_T28)

    xp = jnp.pad(x, ((0, 0), (1, 0), (0, 4))).reshape(1, N28)

    conv_ws = [w00, w02, w04, w06, w08, w10, w20, w22, w24, w26, w28, w30]
    conv_bs = [w01, w03, w05, w07, w09, w11, w21, w23, w25, w27, w29, w31]
    lin_ws = [w12, w14, w16, w18]
    lin_bs = [w13, w15, w17, w19]

    wblocks = []
    for (ci, co), w in zip(_CONVS, conv_ws):
        wblocks.append(jnp.pad(w, ((0, 0), (0, _rpad(co) - co), (0, 32 - ci)))
                       .reshape(9 * _rpad(co), 32))
    for w in lin_ws:
        wblocks.append(jnp.pad(w, ((0, 0), (0, 32 - w.shape[1]))))
    wpack = jnp.concatenate(wblocks, axis=0)
    bpack = jnp.concatenate(
        [jnp.pad(b, ((0, 32 - b.shape[0]), (0, 0))) for b in conv_bs + lin_bs],
        axis=1)

    consts = [
        jnp.asarray(_gap_mask(_T28, B)), jnp.asarray(_gap_mask(_T14, B)),
        jnp.asarray(_gap_mask(_T7, B)),
        jnp.asarray(_pool_sel(_T28, _T14)), jnp.asarray(_pool_sel(_T14, _T7)),
        jnp.asarray(_upsample(_T7, _T14)), jnp.asarray(_upsample(_T14, _T28)),
        jnp.asarray(_avg_mat(B)), jnp.asarray(_bc_mat(B)),
    ]
    args = [xp] + consts + [wpack, bpack]

    buf_w = 2 * _HALO + N28

    enc_t, loss = pl.pallas_call(
        _ae_kernel,
        grid=(1,),
        in_specs=[pl.BlockSpec(a.shape, _zero_map(a.ndim)) for a in args],
        out_specs=(pl.BlockSpec((B, 16), lambda i: (0, 0)),
                   pl.BlockSpec((1, 1), lambda i: (0, 0))),
        out_shape=(jax.ShapeDtypeStruct((B, 16), jnp.float32),
                   jax.ShapeDtypeStruct((1, 1), jnp.float32)),
        scratch_shapes=[
            pltpu.VMEM((32, buf_w), jnp.float32),
            pltpu.VMEM((32, buf_w), jnp.float32),
        ],
        compiler_params=pltpu.CompilerParams(
            dimension_semantics=("arbitrary",),
            vmem_limit_bytes=32 * 1024 * 1024,
        ),
        cost_estimate=pl.CostEstimate(flops=16_000_000, transcendentals=25_000,
                                      bytes_accessed=3_000_000),
    )(*args)
    return enc_t, loss[0, 0]
